# baseline (speedup 1.0000x reference)
import functools

import jax
import jax.numpy as jnp
from jax.experimental import pallas as pl
from jax.experimental.pallas import tpu as pltpu


def _body(x_ref, w_ref, b_ref, s_ref, od_ref, os_ref, acc_ref,
          *, d, n, k, nc):
    m = w_ref.shape[0]
    c = pl.program_id(1)
    part = jnp.dot(w_ref[...], x_ref[0], preferred_element_type=jnp.float32)

    @pl.when(c == 0)
    def _():
        acc_ref[...] = part

    @pl.when(c > 0)
    def _():
        acc_ref[...] = acc_ref[...] + part

    @pl.when(c == nc - 1)
    def _():
        Y = acc_ref[...] + b_ref[...]
        sub = jax.lax.broadcasted_iota(jnp.int32, (m, n), 0)
        s_row = s_ref[0]
        s_col = jnp.transpose(jnp.broadcast_to(s_row, (8, n)), (1, 0))[:, 0:1]
        s_b = jnp.broadcast_to(s_row, (m, n))
        G = jnp.where(sub < d, jnp.maximum(Y, 0.0),
                      jnp.where(sub == d, s_b, 0.0))

        isub = jax.lax.broadcasted_iota(jnp.int32, (n, n), 0)
        jlan = jax.lax.broadcasted_iota(jnp.int32, (n, n), 1)
        sr = jnp.broadcast_to(s_row, (n, n))
        sc = jnp.broadcast_to(s_col, (n, n))
        cmp = (sr > sc) | ((sr == sc) & (jlan < isub))
        rank = jnp.sum(cmp.astype(jnp.float32), axis=1, keepdims=True)

        kf = jax.lax.broadcasted_iota(jnp.int32, (n, n), 1).astype(jnp.float32)
        PT = (jnp.broadcast_to(rank, (n, n)) == kf).astype(jnp.float32)
        O = jnp.dot(G, PT, preferred_element_type=jnp.float32)

        kmask = jax.lax.broadcasted_iota(jnp.int32, (m, n), 1) < k
        Om = jnp.where(kmask, O, 0.0)
        sq = jnp.sum(Om * Om, axis=1, keepdims=True)
        den = jnp.maximum(jnp.sqrt(sq), 1e-12)
        desc = O[0:d, :] / den[0:d, :]
        od_ref[0] = desc[:, 0:k]
        os_ref[0] = O[d:d + 1, 0:k]


def kernel(features, att_w, att_b, proj_w, proj_b, bn_gamma, bn_beta,
           bn_mean, bn_var, num_keypoints):
    B, C, H, W = features.shape
    D = proj_w.shape[0]
    N = H * W
    K = min(1000, N)
    eps = 1e-5
    M = D + 8
    NC = 3
    CB = C // NC

    x = features.reshape(B, C, N)
    att = jax.nn.sigmoid(jnp.einsum('bchw,oc->bohw', features, att_w)
                         + att_b[None, :, None, None])
    s3 = att.reshape(B, 1, N)
    scale = bn_gamma / jnp.sqrt(bn_var + eps)
    w_loc = proj_w * scale[:, None]
    b_loc = (proj_b - bn_mean) * scale + bn_beta
    w_all = jnp.concatenate(
        [w_loc, jnp.zeros((M - D, C), jnp.float32)], axis=0)
    b_all = jnp.concatenate(
        [b_loc, jnp.zeros((M - D,), jnp.float32)], axis=0)
    b_all = jnp.broadcast_to(b_all[:, None], (M, N))

    body = functools.partial(_body, d=D, n=N, k=K, nc=NC)
    out = pl.pallas_call(
        body,
        grid=(B, NC),
        in_specs=[
            pl.BlockSpec((1, CB, N), lambda b, c: (b, c, 0)),
            pl.BlockSpec((M, CB), lambda b, c: (0, c)),
            pl.BlockSpec((M, N), lambda b, c: (0, 0)),
            pl.BlockSpec((1, 1, N), lambda b, c: (b, 0, 0)),
        ],
        out_specs=[
            pl.BlockSpec((1, D, K), lambda b, c: (b, 0, 0)),
            pl.BlockSpec((1, 1, K), lambda b, c: (b, 0, 0)),
        ],
        out_shape=[
            jax.ShapeDtypeStruct((B, D, K), jnp.float32),
            jax.ShapeDtypeStruct((B, 1, K), jnp.float32),
        ],
        scratch_shapes=[pltpu.VMEM((M, N), jnp.float32)],
        compiler_params=pltpu.CompilerParams(
            dimension_semantics=("parallel", "arbitrary")),
    )(x, w_all, b_all, s3)

    local_desc, scores3 = out
    return (local_desc, scores3[:, 0, :])


# speedup vs baseline: 1.7330x; 1.7330x over previous
import functools

import jax
import jax.numpy as jnp
from jax.experimental import pallas as pl
from jax.experimental.pallas import tpu as pltpu


def _body(x_ref, w_ref, b_ref, s_ref, od_ref, os_ref, *, d, n, k):
    m = w_ref.shape[0]
    Xt = x_ref[0]
    Yt = jax.lax.dot_general(Xt, w_ref[...], (((1,), (1,)), ((), ())),
                             preferred_element_type=jnp.float32)
    Yt = Yt + b_ref[0:1, :]

    lane_m = jax.lax.broadcasted_iota(jnp.int32, (n, m), 1)
    s_row = s_ref[pl.ds(pl.program_id(0), 1), :]
    s_col = jnp.transpose(jnp.broadcast_to(s_row, (8, n)), (1, 0))[:, 0:1]
    s_cb = jnp.broadcast_to(s_col, (n, m))
    Gt = jnp.where(lane_m < d, jnp.maximum(Yt, 0.0),
                   jnp.where(lane_m == d, s_cb, 0.0))

    jsub = jax.lax.broadcasted_iota(jnp.int32, (n, n), 0)
    ilan = jax.lax.broadcasted_iota(jnp.int32, (n, n), 1)
    sj = jnp.broadcast_to(s_col, (n, n))
    si = jnp.broadcast_to(s_row, (n, n))
    cmp = (sj > si) | ((sj == si) & (jsub < ilan))
    rank = jnp.sum(cmp.astype(jnp.float32), axis=0, keepdims=True)

    ksub = jax.lax.broadcasted_iota(jnp.int32, (n, n), 0).astype(jnp.float32)
    P = (jnp.broadcast_to(rank, (n, n)) == ksub).astype(jnp.float32)
    Ot = jnp.dot(P, Gt, preferred_element_type=jnp.float32)

    ksel = jax.lax.broadcasted_iota(jnp.int32, (n, m), 0) < k
    Om = jnp.where(ksel, Ot, 0.0)
    sq = jnp.sum(Om * Om, axis=0, keepdims=True)
    den = jnp.maximum(jnp.sqrt(sq), 1e-12)
    On = jnp.where(lane_m < d, Ot / den, Ot)
    T = jnp.transpose(On, (1, 0))
    od_ref[0] = T[0:d, 0:k]
    os_ref[0] = T[d:d + 1, 0:k]


def kernel(features, att_w, att_b, proj_w, proj_b, bn_gamma, bn_beta,
           bn_mean, bn_var, num_keypoints):
    B, C, H, W = features.shape
    D = proj_w.shape[0]
    N = H * W
    K = min(1000, N)
    eps = 1e-5
    M = D + 8

    xt = jnp.transpose(features.reshape(B, C, N), (0, 2, 1))
    att = jax.nn.sigmoid(jnp.einsum('bchw,oc->bohw', features, att_w)
                         + att_b[None, :, None, None])
    s2d = att.reshape(B, N)
    scale = bn_gamma / jnp.sqrt(bn_var + eps)
    w_loc = proj_w * scale[:, None]
    b_loc = (proj_b - bn_mean) * scale + bn_beta
    w_all = jnp.concatenate(
        [w_loc, att_w, jnp.zeros((M - D - 1, C), jnp.float32)], axis=0)
    b_all = jnp.concatenate(
        [b_loc, jnp.zeros((M - D,), jnp.float32)], axis=0)
    b_all = jnp.broadcast_to(b_all[None, :], (8, M))

    body = functools.partial(_body, d=D, n=N, k=K)
    out = pl.pallas_call(
        body,
        grid=(B,),
        in_specs=[
            pl.BlockSpec((1, N, C), lambda b: (b, 0, 0)),
            pl.BlockSpec((M, C), lambda b: (0, 0)),
            pl.BlockSpec((8, M), lambda b: (0, 0)),
            pl.BlockSpec((B, N), lambda b: (0, 0)),
        ],
        out_specs=[
            pl.BlockSpec((1, D, K), lambda b: (b, 0, 0)),
            pl.BlockSpec((1, 1, K), lambda b: (b, 0, 0)),
        ],
        out_shape=[
            jax.ShapeDtypeStruct((B, D, K), jnp.float32),
            jax.ShapeDtypeStruct((B, 1, K), jnp.float32),
        ],
        compiler_params=pltpu.CompilerParams(
            dimension_semantics=("parallel",)),
    )(xt, w_all, b_all, s2d)

    local_desc, scores3 = out
    return (local_desc, scores3[:, 0, :])
